# baseline (speedup 1.0000x reference)
import functools

import jax
import jax.numpy as jnp
from jax import lax
from jax.experimental import pallas as pl
from jax.experimental.pallas import tpu as pltpu
from jax.experimental.pallas import tpu_sc as plsc

_R = 128
_C = 32768
_G = 256
_GPR = _C // _G
_N = _R * _GPR

_CR = 8
_CC = 4096
_GPC = _CC // _G
_NCH = 4
_GPW = 512

_mesh = plsc.VectorSubcoreMesh(core_axis_name="c", subcore_axis_name="s")


_GDN = lax.GatherDimensionNumbers(
    offset_dims=(), collapsed_slice_dims=(0,), start_index_map=(0,))


def _take(v, perm):
    return lax.gather(v, perm[:, None], _GDN, (1,),
                      mode=lax.GatherScatterMode.PROMISE_IN_BOUNDS)


@functools.partial(
    pl.kernel,
    mesh=_mesh,
    out_type=jax.ShapeDtypeStruct((_N,), jnp.int32),
    scratch_types=[
        pltpu.VMEM((_CR, _CC), jnp.float32),
        pltpu.VMEM((_CR, _CC), jnp.float32),
        pltpu.VMEM((_GPW,), jnp.int32),
        pltpu.VMEM((_GPW,), jnp.int32),
        pltpu.SemaphoreType.DMA,
        pltpu.SemaphoreType.DMA,
        pltpu.SemaphoreType.DMA,
    ],
    compiler_params=pltpu.CompilerParams(needs_layout_passes=False),
)
def _sc_idx(x_hbm, idx_hbm, in0, in1, idxbuf, posbuf, si0, si1, so):
    ins, isems = [in0, in1], [si0, si1]
    wid = lax.axis_index("s") * 2 + lax.axis_index("c")
    r0 = (wid % 16) * _CR
    gc0 = (wid // 16) * (_NCH * _GPC)
    lane = lax.iota(jnp.int32, 16)
    perms = [jnp.bitwise_xor(lane, s) for s in (1, 2, 4, 8)]

    def pos_body(i, _):
        p = i * 16 + lane
        gpos = (r0 + jnp.right_shift(p, 6)) * _GPR + gc0 + jnp.bitwise_and(p, 63)
        posbuf[pl.ds(i * 16, 16)] = gpos
        return _

    lax.fori_loop(0, _GPW // 16, pos_body, None)

    def make_group(buf, r):
        def group_body(gc, results):
            base = gc * _G
            vs = [buf[r, pl.ds(base + j * 16, 16)] for j in range(16)]
            js = [jnp.full((16,), j, jnp.int32) for j in range(16)]
            while len(vs) > 1:
                nv, nj = [], []
                for i in range(0, len(vs), 2):
                    tk = vs[i + 1] > vs[i]
                    nv.append(jnp.maximum(vs[i], vs[i + 1]))
                    nj.append(jnp.where(tk, js[i + 1], js[i]))
                vs, js = nv, nj
            cur = vs[0]
            idx = js[0] * 16 + lane
            for perm in perms:
                pc = _take(cur, perm)
                pi = _take(idx, perm)
                take = (pc > cur) | ((pc == cur) & (pi < idx))
                cur = jnp.where(take, pc, cur)
                idx = jnp.where(take, pi, idx)
            return jnp.where(lane == gc, idx, results)
        return group_body

    def start_in(c, b):
        cc = (gc0 + c * _GPC) * _G
        return pltpu.async_copy(
            x_hbm.at[pl.ds(r0, _CR), pl.ds(cc, _CC)], ins[b], isems[b])

    in_h = {0: start_in(0, 0)}
    for c in range(_NCH):
        b = c & 1
        if c + 1 < _NCH:
            in_h[c + 1] = start_in(c + 1, 1 - b)
        in_h[c].wait()
        for r in range(_CR):
            res = plsc.parallel_loop(
                0, _GPC, carry=jnp.zeros((16,), jnp.int32), unroll=2)(
                    make_group(ins[b], r))
            idxbuf[pl.ds(r * (_GPW // _CR) + c * _GPC, 16)] = res
    pltpu.async_copy(idxbuf, idx_hbm.at[posbuf], so).wait()


_BR = 8


def _tc_body(idx_ref, o_ref):
    iota = lax.broadcasted_iota(jnp.int32, (_BR, _G), 1).astype(jnp.float32)
    idx_f = idx_ref[...].astype(jnp.float32)
    for k in range(_GPR):
        col = idx_f[:, k:k + 1]
        o_ref[:, k * _G:(k + 1) * _G] = jnp.where(iota == col, 1.0, 0.0)


def kernel(x):
    idx = _sc_idx(x)
    idx2d = idx.reshape(_R, _GPR)
    return pl.pallas_call(
        _tc_body,
        grid=(_R // _BR,),
        in_specs=[pl.BlockSpec((_BR, _GPR), lambda i: (i, 0))],
        out_specs=pl.BlockSpec((_BR, _C), lambda i: (i, 0)),
        out_shape=jax.ShapeDtypeStruct((_R, _C), jnp.float32),
    )(idx2d)


# speedup vs baseline: 1.0459x; 1.0459x over previous
import functools

import jax
import jax.numpy as jnp
from jax import lax
from jax.experimental import pallas as pl
from jax.experimental.pallas import tpu as pltpu
from jax.experimental.pallas import tpu_sc as plsc

_R = 128
_C = 32768
_G = 256
_GPR = _C // _G
_N = _R * _GPR

_CR = 8
_CC = 4096
_GPC = _CC // _G
_NCH = 4
_GPW = 512

_mesh = plsc.VectorSubcoreMesh(core_axis_name="c", subcore_axis_name="s")


_GDN = lax.GatherDimensionNumbers(
    offset_dims=(), collapsed_slice_dims=(0,), start_index_map=(0,))


def _take(v, perm):
    return lax.gather(v, perm[:, None], _GDN, (1,),
                      mode=lax.GatherScatterMode.PROMISE_IN_BOUNDS)


@functools.partial(
    pl.kernel,
    mesh=_mesh,
    out_type=jax.ShapeDtypeStruct((_N,), jnp.int32),
    scratch_types=[
        pltpu.VMEM((_CR, _CC), jnp.float32),
        pltpu.VMEM((_CR, _CC), jnp.float32),
        pltpu.VMEM((_GPW,), jnp.int32),
        pltpu.VMEM((_GPW,), jnp.int32),
        pltpu.SemaphoreType.DMA,
        pltpu.SemaphoreType.DMA,
        pltpu.SemaphoreType.DMA,
    ],
    compiler_params=pltpu.CompilerParams(needs_layout_passes=False),
)
def _sc_idx(x_hbm, idx_hbm, in0, in1, idxbuf, posbuf, si0, si1, so):
    ins, isems = [in0, in1], [si0, si1]
    wid = lax.axis_index("s") * 2 + lax.axis_index("c")
    r0 = (wid % 16) * _CR
    gc0 = (wid // 16) * (_NCH * _GPC)
    lane = lax.iota(jnp.int32, 16)
    perms = [jnp.bitwise_xor(lane, s) for s in (1, 2, 4, 8)]

    def pos_body(i, _):
        p = i * 16 + lane
        gpos = (r0 + jnp.right_shift(p, 6)) * _GPR + gc0 + jnp.bitwise_and(p, 63)
        posbuf[pl.ds(i * 16, 16)] = gpos
        return _

    lax.fori_loop(0, _GPW // 16, pos_body, None)

    lane0 = lane == 0

    def make_group(buf, c):
        def group_body(g):
            row = jnp.right_shift(g, 4)
            gc = jnp.bitwise_and(g, 15)
            base = gc * _G
            vs = [buf[row, pl.ds(base + j * 16, 16)] for j in range(16)]
            js = [jnp.full((16,), j, jnp.int32) for j in range(16)]
            while len(vs) > 1:
                nv, nj = [], []
                for i in range(0, len(vs), 2):
                    tk = vs[i + 1] > vs[i]
                    nv.append(jnp.maximum(vs[i], vs[i + 1]))
                    nj.append(jnp.where(tk, js[i + 1], js[i]))
                vs, js = nv, nj
            cur = vs[0]
            idx = js[0] * 16 + lane
            for perm in perms:
                pc = _take(cur, perm)
                pi = _take(idx, perm)
                take = (pc > cur) | ((pc == cur) & (pi < idx))
                cur = jnp.where(take, pc, cur)
                idx = jnp.where(take, pi, idx)
            p = row * (_GPW // _CR) + c * _GPC + gc
            plsc.store_scatter(idxbuf, [jnp.full((16,), 0, jnp.int32) + p],
                               idx, mask=lane0)
        return group_body

    def start_in(c, b):
        cc = (gc0 + c * _GPC) * _G
        return pltpu.async_copy(
            x_hbm.at[pl.ds(r0, _CR), pl.ds(cc, _CC)], ins[b], isems[b])

    in_h = {0: start_in(0, 0)}
    for c in range(_NCH):
        b = c & 1
        if c + 1 < _NCH:
            in_h[c + 1] = start_in(c + 1, 1 - b)
        in_h[c].wait()
        plsc.parallel_loop(0, _CR * _GPC, unroll=8)(make_group(ins[b], c))
    pltpu.async_copy(idxbuf, idx_hbm.at[posbuf], so).wait()


_BR = 8


def _tc_body(idx_ref, o_ref):
    iota = lax.broadcasted_iota(jnp.int32, (_BR, _G), 1).astype(jnp.float32)
    idx_f = idx_ref[...].astype(jnp.float32)
    for k in range(_GPR):
        col = idx_f[:, k:k + 1]
        o_ref[:, k * _G:(k + 1) * _G] = jnp.where(iota == col, 1.0, 0.0)


def kernel(x):
    idx = _sc_idx(x)
    idx2d = idx.reshape(_R, _GPR)
    return pl.pallas_call(
        _tc_body,
        grid=(_R // _BR,),
        in_specs=[pl.BlockSpec((_BR, _GPR), lambda i: (i, 0))],
        out_specs=pl.BlockSpec((_BR, _C), lambda i: (i, 0)),
        out_shape=jax.ShapeDtypeStruct((_R, _C), jnp.float32),
    )(idx2d)


# speedup vs baseline: 1.0499x; 1.0038x over previous
import functools

import jax
import jax.numpy as jnp
from jax import lax
from jax.experimental import pallas as pl
from jax.experimental.pallas import tpu as pltpu
from jax.experimental.pallas import tpu_sc as plsc

_R = 128
_C = 32768
_G = 256
_GPR = _C // _G
_N = _R * _GPR

_CR = 8
_CC = 4096
_GPC = _CC // _G
_NCH = 4
_GPW = 512

_mesh = plsc.VectorSubcoreMesh(core_axis_name="c", subcore_axis_name="s")


_GDN = lax.GatherDimensionNumbers(
    offset_dims=(), collapsed_slice_dims=(0,), start_index_map=(0,))


def _take(v, perm):
    return lax.gather(v, perm[:, None], _GDN, (1,),
                      mode=lax.GatherScatterMode.PROMISE_IN_BOUNDS)


@functools.partial(
    pl.kernel,
    mesh=_mesh,
    out_type=jax.ShapeDtypeStruct((_N,), jnp.int32),
    scratch_types=[
        pltpu.VMEM((_CR, _CC), jnp.float32),
        pltpu.VMEM((_CR, _CC), jnp.float32),
        pltpu.VMEM((_GPW,), jnp.int32),
        pltpu.VMEM((_GPW,), jnp.int32),
        pltpu.SemaphoreType.DMA,
        pltpu.SemaphoreType.DMA,
        pltpu.SemaphoreType.DMA,
    ],
    compiler_params=pltpu.CompilerParams(needs_layout_passes=False),
)
def _sc_idx(x_hbm, idx_hbm, in0, in1, idxbuf, posbuf, si0, si1, so):
    ins, isems = [in0, in1], [si0, si1]
    wid = lax.axis_index("s") * 2 + lax.axis_index("c")
    r0 = (wid % 16) * _CR
    gc0 = (wid // 16) * (_NCH * _GPC)
    lane = lax.iota(jnp.int32, 16)
    perms = [jnp.bitwise_xor(lane, s) for s in (1, 2, 4, 8)]

    def pos_body(i, _):
        p = i * 16 + lane
        gpos = (r0 + jnp.right_shift(p, 6)) * _GPR + gc0 + jnp.bitwise_and(p, 63)
        posbuf[pl.ds(i * 16, 16)] = gpos
        return _

    lax.fori_loop(0, _GPW // 16, pos_body, None)

    lane0 = lane == 0

    def make_group(buf, c):
        def group_body(g):
            row = jnp.right_shift(g, 4)
            gc = jnp.bitwise_and(g, 15)
            base = gc * _G
            vs = [buf[row, pl.ds(base + j * 16, 16)] for j in range(16)]
            js = [jnp.full((16,), j, jnp.int32) for j in range(16)]
            while len(vs) > 1:
                nv, nj = [], []
                for i in range(0, len(vs), 2):
                    tk = vs[i + 1] > vs[i]
                    nv.append(jnp.maximum(vs[i], vs[i + 1]))
                    nj.append(jnp.where(tk, js[i + 1], js[i]))
                vs, js = nv, nj
            cur = vs[0]
            idx = js[0] * 16 + lane
            for perm in perms:
                pc = _take(cur, perm)
                pi = _take(idx, perm)
                take = (pc > cur) | ((pc == cur) & (pi < idx))
                cur = jnp.where(take, pc, cur)
                idx = jnp.where(take, pi, idx)
            p = row * (_GPW // _CR) + c * _GPC + gc
            plsc.store_scatter(idxbuf, [jnp.full((16,), 0, jnp.int32) + p],
                               idx, mask=lane0)
        return group_body

    def start_in(c, b):
        cc = (gc0 + c * _GPC) * _G
        return pltpu.async_copy(
            x_hbm.at[pl.ds(r0, _CR), pl.ds(cc, _CC)], ins[b], isems[b])

    in_h = {0: start_in(0, 0)}
    for c in range(_NCH):
        b = c & 1
        if c + 1 < _NCH:
            in_h[c + 1] = start_in(c + 1, 1 - b)
        in_h[c].wait()
        plsc.parallel_loop(0, _CR * _GPC, unroll=16)(make_group(ins[b], c))
    pltpu.async_copy(idxbuf, idx_hbm.at[posbuf], so).wait()


_BR = 8


def _tc_body(idx_ref, o_ref):
    iota = lax.broadcasted_iota(jnp.int32, (_BR, _G), 1).astype(jnp.float32)
    idx_f = idx_ref[...].astype(jnp.float32)
    for k in range(_GPR):
        col = idx_f[:, k:k + 1]
        o_ref[:, k * _G:(k + 1) * _G] = jnp.where(iota == col, 1.0, 0.0)


def kernel(x):
    idx = _sc_idx(x)
    idx2d = idx.reshape(_R, _GPR)
    return pl.pallas_call(
        _tc_body,
        grid=(_R // _BR,),
        in_specs=[pl.BlockSpec((_BR, _GPR), lambda i: (i, 0))],
        out_specs=pl.BlockSpec((_BR, _C), lambda i: (i, 0)),
        out_shape=jax.ShapeDtypeStruct((_R, _C), jnp.float32),
    )(idx2d)


# speedup vs baseline: 1.1960x; 1.1392x over previous
import functools

import jax
import jax.numpy as jnp
from jax import lax
from jax.experimental import pallas as pl
from jax.experimental.pallas import tpu as pltpu
from jax.experimental.pallas import tpu_sc as plsc

_R = 128
_C = 32768
_G = 256
_CR = 8
_CC = 2048
_NSB = 4
_NCH = 8

_mesh = plsc.VectorSubcoreMesh(core_axis_name="c", subcore_axis_name="s")


@functools.partial(
    pl.kernel,
    mesh=_mesh,
    out_type=jax.ShapeDtypeStruct((_R, _C), jnp.float32),
    scratch_types=[
        pltpu.VMEM((_CR, _CC), jnp.float32),
        pltpu.VMEM((_CR, _CC), jnp.float32),
        pltpu.VMEM((_CR, _CC), jnp.float32),
        pltpu.VMEM((_CR, _CC), jnp.float32),
        pltpu.SemaphoreType.DMA,
        pltpu.SemaphoreType.DMA,
        pltpu.SemaphoreType.DMA,
        pltpu.SemaphoreType.DMA,
    ],
    compiler_params=pltpu.CompilerParams(needs_layout_passes=False),
)
def _sc_kernel(x_hbm, out_hbm, in0, in1, ou0, ou1, si0, si1, so0, so1):
    ins, outs = [in0, in1], [ou0, ou1]
    isems, osems = [si0, si1], [so0, so1]
    wid = lax.axis_index("s") * 2 + lax.axis_index("c")
    r0 = (wid % 16) * _CR
    c0 = (wid // 16) * (_NCH * _CC)
    lane = lax.iota(jnp.int32, 16)
    zeros = jnp.zeros((16,), jnp.float32)
    ones = jnp.ones((16,), jnp.float32)
    neginf = jnp.full((16,), -jnp.inf, jnp.float32)

    row_idx = jnp.bitwise_and(lane, 7)
    colb = [(jnp.right_shift(lane, 3) + 2 * b) * _G for b in range(_NSB)]

    def zero_body(i, _):
        for r in range(_CR):
            ou0[r, pl.ds(i * 16, 16)] = zeros
            ou1[r, pl.ds(i * 16, 16)] = zeros
        return _

    lax.fori_loop(0, _CC // 16, zero_body, None)

    def make_scan(buf):
        def scan_elems(e, carry):
            curs, idxs = carry
            new_curs, new_idxs = [], []
            for b in range(_NSB):
                v = plsc.load_gather(buf, [row_idx, colb[b] + e])
                upd = v > curs[b]
                new_curs.append(jnp.maximum(curs[b], v))
                new_idxs.append(jnp.where(upd, e, idxs[b]))
            return tuple(new_curs), tuple(new_idxs)
        return scan_elems

    def start_in(c, b):
        cc = c0 + c * _CC
        return pltpu.async_copy(
            x_hbm.at[pl.ds(r0, _CR), pl.ds(cc, _CC)], ins[b], isems[b])

    def start_out(c, b):
        cc = c0 + c * _CC
        return pltpu.async_copy(
            outs[b], out_hbm.at[pl.ds(r0, _CR), pl.ds(cc, _CC)], osems[b])

    in_h = {0: start_in(0, 0)}
    out_h = {}
    prev_ones = [None, None]
    init = (tuple(neginf for _ in range(_NSB)),
            tuple(jnp.zeros((16,), jnp.int32) for _ in range(_NSB)))
    for c in range(_NCH):
        b = c & 1
        if c + 1 < _NCH:
            in_h[c + 1] = start_in(c + 1, 1 - b)
        in_h[c].wait()
        if c >= 2:
            out_h[c - 2].wait()
            for oc in prev_ones[b]:
                plsc.store_scatter(outs[b], [row_idx, oc], zeros)
        _, idxs = lax.fori_loop(0, _G, make_scan(ins[b]), init, unroll=8)
        onecols = [colb[k] + idxs[k] for k in range(_NSB)]
        for oc in onecols:
            plsc.store_scatter(outs[b], [row_idx, oc], ones)
        out_h[c] = start_out(c, b)
        prev_ones[b] = onecols
    out_h[_NCH - 2].wait()
    out_h[_NCH - 1].wait()


def kernel(x):
    return _sc_kernel(x)
